# baseline (speedup 1.0000x reference)
import functools

import jax
import jax.numpy as jnp
from jax.experimental import pallas as pl
from jax.experimental.pallas import tpu as pltpu

B = 4
N = 10000
E = 320000
DV = 128
DE = 16
DC = 32

VCH = 2000
ECH = 8000
NVB = N // VCH
NEB = E // ECH
SPB = NVB + NEB


def _tc_kernel(ctx_ref, v_ref, e_ref, w_ref, b_ref, out_ref,
               acc_v, acc_e, agg_v, agg_e):
    bi = pl.program_id(0)
    j = pl.program_id(1)

    @pl.when(j == 0)
    def _init():
        acc_v[...] = jnp.zeros_like(acc_v)
        acc_e[...] = jnp.zeros_like(acc_e)

    @pl.when(j < NVB)
    def _vstep():
        acc_v[...] += jnp.sum(v_ref[0].reshape(VCH // 8, 8, DV), axis=0)

    @pl.when(j >= NVB)
    def _estep():
        acc_e[...] += jnp.sum(e_ref[0].reshape(ECH // 8, 8, DE), axis=0)

    @pl.when(j == SPB - 1)
    def _flush():
        agg_v[bi] = jnp.sum(acc_v[...], axis=0)
        agg_e[bi] = jnp.sum(acc_e[...], axis=0)

    @pl.when((bi == B - 1) & (j == SPB - 1))
    def _final():
        v_agg = agg_v[...] * (1.0 / N)
        e_agg = agg_e[...] * (1.0 / E)
        ctx = ctx_ref[...][:, 0, :]
        w = w_ref[...]
        out = (
            jnp.dot(ctx, w[:DC], preferred_element_type=jnp.float32)
            + jnp.dot(v_agg, w[DC:DC + DV], preferred_element_type=jnp.float32)
            + jnp.dot(e_agg, w[DC + DV:], preferred_element_type=jnp.float32)
            + b_ref[...][None, :]
        )
        out_ref[...] = out[:, None, :]


@jax.jit
def kernel(context, vertex_data, edge_data, W, b):
    grid = (B, SPB)
    return pl.pallas_call(
        _tc_kernel,
        grid=grid,
        in_specs=[
            pl.BlockSpec((B, 1, DC), lambda i, j: (0, 0, 0)),
            pl.BlockSpec((1, VCH, DV), lambda i, j: (i, jnp.minimum(j, NVB - 1), 0)),
            pl.BlockSpec((1, ECH, DE), lambda i, j: (i, jnp.maximum(j - NVB, 0), 0)),
            pl.BlockSpec((DC + DV + DE, DC), lambda i, j: (0, 0)),
            pl.BlockSpec((DC,), lambda i, j: (0,)),
        ],
        out_specs=pl.BlockSpec((B, 1, DC), lambda i, j: (0, 0, 0)),
        out_shape=jax.ShapeDtypeStruct((B, 1, DC), jnp.float32),
        scratch_shapes=[
            pltpu.VMEM((8, DV), jnp.float32),
            pltpu.VMEM((8, DE), jnp.float32),
            pltpu.VMEM((B, DV), jnp.float32),
            pltpu.VMEM((B, DE), jnp.float32),
        ],
        compiler_params=pltpu.CompilerParams(
            dimension_semantics=("arbitrary", "arbitrary"),
        ),
    )(context, vertex_data, edge_data, W, b)


# speedup vs baseline: 1.1387x; 1.1387x over previous
import functools

import jax
import jax.numpy as jnp
from jax import lax
from jax.experimental import pallas as pl
from jax.experimental.pallas import tpu as pltpu
from jax.experimental.pallas import tpu_sc as plsc

B = 4
N = 10000
E = 320000
DV = 128
DE = 16
DC = 32

NC = 2
NS = 16
NW = NC * NS
WPB = NW // B

VSH = 1248
VCH = 208
NVCH = VSH // VCH
VTAIL = N - WPB * VSH
ESH = E // WPB
ECH = 1600
NECH = ESH // ECH

_sc_mesh = plsc.VectorSubcoreMesh(core_axis_name="c", subcore_axis_name="s")


@functools.partial(
    pl.kernel,
    out_type=(
        jax.ShapeDtypeStruct((B, WPB, DV), jnp.float32),
        jax.ShapeDtypeStruct((B, WPB, DE), jnp.float32),
    ),
    mesh=_sc_mesh,
    scratch_types=[
        pltpu.VMEM((2, VCH, DV), jnp.float32),
        pltpu.VMEM((2, ECH, DE), jnp.float32),
        pltpu.VMEM((DV,), jnp.float32),
        pltpu.VMEM((DE,), jnp.float32),
        pltpu.SemaphoreType.DMA,
        pltpu.SemaphoreType.DMA,
    ],
    compiler_params=pltpu.CompilerParams(use_tc_tiling_on_sc=False),
)
def _sc_partial(v_hbm, e_hbm, vout_hbm, eout_hbm,
                vbuf, ebuf, vst, est, sem0, sem1):
    wid = lax.axis_index("s") * NC + lax.axis_index("c")
    bi = wid // WPB
    ji = wid % WPB
    sems = (sem0, sem1)

    vbase = ji * VSH
    for c in range(2):
        pltpu.async_copy(
            v_hbm.at[bi, pl.ds(vbase + c * VCH, VCH), :], vbuf.at[c], sems[c])

    def _vchunk(slot, accs, nrows):
        def body(i, a):
            return tuple(
                a[k] + vbuf[slot, i, pl.ds(k * 16, 16)] for k in range(8))
        return lax.fori_loop(0, nrows, body, accs)

    vaccs = tuple(jnp.zeros((16,), jnp.float32) for _ in range(8))
    for c in range(NVCH):
        slot = c % 2
        pltpu.make_async_copy(
            v_hbm.at[bi, pl.ds(vbase + c * VCH, VCH), :], vbuf.at[slot],
            sems[slot]).wait()
        vaccs = _vchunk(slot, vaccs, VCH)
        if c + 2 < NVCH:
            pltpu.async_copy(
                v_hbm.at[bi, pl.ds(vbase + (c + 2) * VCH, VCH), :],
                vbuf.at[slot], sems[slot])

    for k in range(8):
        vst[pl.ds(k * 16, 16)] = vaccs[k]

    @pl.when(ji == WPB - 1)
    def _vtail():
        pltpu.sync_copy(
            v_hbm.at[bi, pl.ds(WPB * VSH, VTAIL), :],
            vbuf.at[0, pl.ds(0, VTAIL), :])
        taccs = _vchunk(0, tuple(jnp.zeros((16,), jnp.float32)
                                 for _ in range(8)), VTAIL)
        for k in range(8):
            vst[pl.ds(k * 16, 16)] = vst[pl.ds(k * 16, 16)] + taccs[k]

    pltpu.sync_copy(vst, vout_hbm.at[bi, ji])

    ebase = ji * ESH
    for c in range(2):
        pltpu.async_copy(
            e_hbm.at[bi, pl.ds(ebase + c * ECH, ECH), :], ebuf.at[c], sems[c])

    def _echunk(slot, accs):
        def body(i, a):
            return tuple(a[k] + ebuf[slot, i * 8 + k, :] for k in range(8))
        return lax.fori_loop(0, ECH // 8, body, accs)

    eaccs = tuple(jnp.zeros((16,), jnp.float32) for _ in range(8))
    for c in range(NECH):
        slot = c % 2
        pltpu.make_async_copy(
            e_hbm.at[bi, pl.ds(ebase + c * ECH, ECH), :], ebuf.at[slot],
            sems[slot]).wait()
        eaccs = _echunk(slot, eaccs)
        if c + 2 < NECH:
            pltpu.async_copy(
                e_hbm.at[bi, pl.ds(ebase + (c + 2) * ECH, ECH), :],
                ebuf.at[slot], sems[slot])

    esum = eaccs[0]
    for k in range(1, 8):
        esum = esum + eaccs[k]
    est[...] = esum
    pltpu.sync_copy(est, eout_hbm.at[bi, ji])


def _tc_finish(ctx_ref, vp_ref, ep_ref, w_ref, b_ref, out_ref):
    v_agg = jnp.sum(vp_ref[...], axis=1) * (1.0 / N)
    e_agg = jnp.sum(ep_ref[...], axis=1) * (1.0 / E)
    ctx = ctx_ref[...][:, 0, :]
    w = w_ref[...]
    out = (
        jnp.dot(ctx, w[:DC], preferred_element_type=jnp.float32)
        + jnp.dot(v_agg, w[DC:DC + DV], preferred_element_type=jnp.float32)
        + jnp.dot(e_agg, w[DC + DV:], preferred_element_type=jnp.float32)
        + b_ref[...][None, :]
    )
    out_ref[...] = out[:, None, :]


@jax.jit
def kernel(context, vertex_data, edge_data, W, b):
    vp, ep = _sc_partial(vertex_data, edge_data)
    return pl.pallas_call(
        _tc_finish,
        out_shape=jax.ShapeDtypeStruct((B, 1, DC), jnp.float32),
    )(context, vp, ep, W, b)


# speedup vs baseline: 7.7744x; 6.8275x over previous
import jax
import jax.numpy as jnp
from jax.experimental import pallas as pl
from jax.experimental.pallas import tpu as pltpu

B = 4
N = 10000
E = 320000
DV = 128
DE = 16
DC = 32

GT = 2
CT = 2500

VCH = 2000
NVB = N // VCH
ECT = 250
NEB = CT // ECT
NVSTEP = B * NVB
NESTEP = B * GT * NEB
NSTEPS = NVSTEP + NESTEP


def _tc_kernel(ctx_ref, v_ref, e_ref, w_ref, b_ref, out_ref, acc_v, acc_e):
    i = pl.program_id(0)

    @pl.when(i == 0)
    def _init():
        acc_v[...] = jnp.zeros_like(acc_v)
        acc_e[...] = jnp.zeros_like(acc_e)

    @pl.when(i < NVSTEP)
    def _vstep():
        bi = i // NVB
        acc_v[bi] += jnp.sum(v_ref[0].reshape(VCH // 8, 8, 128), axis=0)

    @pl.when(i >= NVSTEP)
    def _estep():
        j = i - NVSTEP
        bi = j // (GT * NEB)
        gi = (j % (GT * NEB)) // NEB
        acc_e[bi, gi] += jnp.sum(e_ref[0, 0], axis=0)

    @pl.when(i == NSTEPS - 1)
    def _final():
        v_agg = jnp.sum(acc_v[...], axis=1) * (1.0 / N)
        e_agg = jnp.sum(acc_e[...], axis=3).reshape(B, DE) * (1.0 / E)
        ctx = ctx_ref[...][:, 0, :]
        w = w_ref[...]
        out = (
            jnp.dot(ctx, w[:DC], preferred_element_type=jnp.float32)
            + jnp.dot(v_agg, w[DC:DC + DV], preferred_element_type=jnp.float32)
            + jnp.dot(e_agg, w[DC + DV:], preferred_element_type=jnp.float32)
            + b_ref[...][None, :]
        )
        out_ref[...] = out[:, None, :]


@jax.jit
def kernel(context, vertex_data, edge_data, W, b):
    e_view = edge_data.reshape(B, CT, 128, GT, 8).transpose(0, 3, 1, 4, 2)
    grid = (NSTEPS,)

    def emap(i):
        j = jnp.maximum(i - NVSTEP, 0)
        return (j // (GT * NEB), (j % (GT * NEB)) // NEB, j % NEB, 0, 0)

    def vmap_(i):
        k = jnp.minimum(i, NVSTEP - 1)
        return (k // NVB, k % NVB, 0)

    return pl.pallas_call(
        _tc_kernel,
        grid=grid,
        in_specs=[
            pl.BlockSpec((B, 1, DC), lambda i: (0, 0, 0)),
            pl.BlockSpec((1, VCH, DV), vmap_),
            pl.BlockSpec((1, 1, ECT, 8, 128), emap),
            pl.BlockSpec((DC + DV + DE, DC), lambda i: (0, 0)),
            pl.BlockSpec((DC,), lambda i: (0,)),
        ],
        out_specs=pl.BlockSpec((B, 1, DC), lambda i: (0, 0, 0)),
        out_shape=jax.ShapeDtypeStruct((B, 1, DC), jnp.float32),
        scratch_shapes=[
            pltpu.VMEM((B, 8, DV), jnp.float32),
            pltpu.VMEM((B, GT, 8, 128), jnp.float32),
        ],
        compiler_params=pltpu.CompilerParams(
            dimension_semantics=("arbitrary",),
        ),
    )(context, vertex_data, e_view, W, b)


# speedup vs baseline: 10.2475x; 1.3181x over previous
import jax
import jax.numpy as jnp
from jax.experimental import pallas as pl
from jax.experimental.pallas import tpu as pltpu

B = 4
N = 10000
E = 320000
DV = 128
DE = 16
DC = 32

GT = 2
CT = 2500

VCH = 2000
NVB = N // VCH
ECT = 625
NEB = CT // ECT
NVSTEP = B * NVB
NESTEP = B * GT * NEB
NSTEPS = NVSTEP + NESTEP


def _tc_kernel(ctx_ref, v_ref, e_ref, w_ref, b_ref, out_ref, acc_v, acc_e):
    i = pl.program_id(0)

    @pl.when(i == 0)
    def _init():
        acc_v[...] = jnp.zeros_like(acc_v)
        acc_e[...] = jnp.zeros_like(acc_e)

    @pl.when(i < NVSTEP)
    def _vstep():
        bi = i // NVB
        acc_v[bi] += jnp.sum(v_ref[0].reshape(VCH // 8, 8, 128), axis=0)

    @pl.when(i >= NVSTEP)
    def _estep():
        j = i - NVSTEP
        bi = j // (GT * NEB)
        gi = (j % (GT * NEB)) // NEB
        acc_e[bi, gi] += jnp.sum(e_ref[0, 0], axis=0)

    @pl.when(i == NSTEPS - 1)
    def _final():
        v_agg = jnp.sum(acc_v[...], axis=1) * (1.0 / N)
        e_agg = jnp.sum(acc_e[...], axis=3).reshape(B, DE) * (1.0 / E)
        ctx = ctx_ref[...][:, 0, :]
        w = w_ref[...]
        out = (
            jnp.dot(ctx, w[:DC], preferred_element_type=jnp.float32)
            + jnp.dot(v_agg, w[DC:DC + DV], preferred_element_type=jnp.float32)
            + jnp.dot(e_agg, w[DC + DV:], preferred_element_type=jnp.float32)
            + b_ref[...][None, :]
        )
        out_ref[...] = out[:, None, :]


@jax.jit
def kernel(context, vertex_data, edge_data, W, b):
    e_view = edge_data.reshape(B, CT, 128, GT, 8).transpose(0, 3, 1, 4, 2)
    grid = (NSTEPS,)

    def emap(i):
        j = jnp.maximum(i - NVSTEP, 0)
        return (j // (GT * NEB), (j % (GT * NEB)) // NEB, j % NEB, 0, 0)

    def vmap_(i):
        k = jnp.minimum(i, NVSTEP - 1)
        return (k // NVB, k % NVB, 0)

    return pl.pallas_call(
        _tc_kernel,
        grid=grid,
        in_specs=[
            pl.BlockSpec((B, 1, DC), lambda i: (0, 0, 0)),
            pl.BlockSpec((1, VCH, DV), vmap_),
            pl.BlockSpec((1, 1, ECT, 8, 128), emap),
            pl.BlockSpec((DC + DV + DE, DC), lambda i: (0, 0)),
            pl.BlockSpec((DC,), lambda i: (0,)),
        ],
        out_specs=pl.BlockSpec((B, 1, DC), lambda i: (0, 0, 0)),
        out_shape=jax.ShapeDtypeStruct((B, 1, DC), jnp.float32),
        scratch_shapes=[
            pltpu.VMEM((B, 8, DV), jnp.float32),
            pltpu.VMEM((B, GT, 8, 128), jnp.float32),
        ],
        compiler_params=pltpu.CompilerParams(
            dimension_semantics=("arbitrary",),
        ),
    )(context, vertex_data, e_view, W, b)


# speedup vs baseline: 13.2314x; 1.2912x over previous
import jax
import jax.numpy as jnp
from jax.experimental import pallas as pl
from jax.experimental.pallas import tpu as pltpu

B = 4
N = 10000
E = 320000
DV = 128
DE = 16
DC = 32

GT = 2
CT = 2500

VCH = 5000
NVB = N // VCH
ECT = 1250
NEB = CT // ECT
NVSTEP = B * NVB
NESTEP = B * GT * NEB
NSTEPS = NVSTEP + NESTEP


def _tc_kernel(ctx_ref, v_ref, e_ref, w_ref, b_ref, out_ref, acc_v, acc_e):
    i = pl.program_id(0)

    @pl.when(i == 0)
    def _init():
        acc_v[...] = jnp.zeros_like(acc_v)
        acc_e[...] = jnp.zeros_like(acc_e)

    @pl.when(i < NVSTEP)
    def _vstep():
        bi = i // NVB
        acc_v[bi] += jnp.sum(v_ref[0].reshape(VCH // 8, 8, 128), axis=0)

    @pl.when(i >= NVSTEP)
    def _estep():
        j = i - NVSTEP
        bi = j // (GT * NEB)
        gi = (j % (GT * NEB)) // NEB
        acc_e[bi, gi] += jnp.sum(e_ref[0, 0], axis=0)

    @pl.when(i == NSTEPS - 1)
    def _final():
        v_agg = jnp.sum(acc_v[...], axis=1) * (1.0 / N)
        e_agg = jnp.sum(acc_e[...], axis=3).reshape(B, DE) * (1.0 / E)
        ctx = ctx_ref[...][:, 0, :]
        w = w_ref[...]
        out = (
            jnp.dot(ctx, w[:DC], preferred_element_type=jnp.float32)
            + jnp.dot(v_agg, w[DC:DC + DV], preferred_element_type=jnp.float32)
            + jnp.dot(e_agg, w[DC + DV:], preferred_element_type=jnp.float32)
            + b_ref[...][None, :]
        )
        out_ref[...] = out[:, None, :]


@jax.jit
def kernel(context, vertex_data, edge_data, W, b):
    e_view = edge_data.reshape(B, CT, 128, GT, 8).transpose(0, 3, 1, 4, 2)
    grid = (NSTEPS,)

    def emap(i):
        j = jnp.maximum(i - NVSTEP, 0)
        return (j // (GT * NEB), (j % (GT * NEB)) // NEB, j % NEB, 0, 0)

    def vmap_(i):
        k = jnp.minimum(i, NVSTEP - 1)
        return (k // NVB, k % NVB, 0)

    return pl.pallas_call(
        _tc_kernel,
        grid=grid,
        in_specs=[
            pl.BlockSpec((B, 1, DC), lambda i: (0, 0, 0)),
            pl.BlockSpec((1, VCH, DV), vmap_),
            pl.BlockSpec((1, 1, ECT, 8, 128), emap),
            pl.BlockSpec((DC + DV + DE, DC), lambda i: (0, 0)),
            pl.BlockSpec((DC,), lambda i: (0,)),
        ],
        out_specs=pl.BlockSpec((B, 1, DC), lambda i: (0, 0, 0)),
        out_shape=jax.ShapeDtypeStruct((B, 1, DC), jnp.float32),
        scratch_shapes=[
            pltpu.VMEM((B, 8, DV), jnp.float32),
            pltpu.VMEM((B, GT, 8, 128), jnp.float32),
        ],
        compiler_params=pltpu.CompilerParams(
            dimension_semantics=("arbitrary",),
        ),
    )(context, vertex_data, e_view, W, b)
